# baseline (speedup 1.0000x reference)
import functools

import jax
import jax.numpy as jnp
from jax import lax
from jax.experimental import pallas as pl
from jax.experimental.pallas import tpu as pltpu
from jax.experimental.pallas import tpu_sc as plsc

D = 64
CW = 64
L = 16


def _build(B):
    info = plsc.get_sparse_core_info()
    NC, NS = info.num_cores, info.num_subcores
    NW = NC * NS
    BW = B // NW
    P = BW // CW
    row_bytes = D * 4

    mesh = plsc.VectorSubcoreMesh(core_axis_name="c", subcore_axis_name="s")

    @functools.partial(
        pl.kernel,
        mesh=mesh,
        out_type=jax.ShapeDtypeStruct((B, 2, D), jnp.float32),
        scratch_types=[
            pltpu.VMEM((BW,), jnp.int32),
            pltpu.VMEM((BW,), jnp.int32),
            pltpu.VMEM((BW,), jnp.int32),
            pltpu.VMEM((3, D), jnp.float32),
            pltpu.VMEM((5, D), jnp.float32),
            pltpu.VMEM((15, D), jnp.float32),
            pltpu.VMEM((2, CW, D), jnp.float32),
            pltpu.VMEM((2, CW, 2, D), jnp.float32),
            pltpu.SemaphoreType.DMA,
            pltpu.SemaphoreType.DMA,
            pltpu.SemaphoreType.DMA,
        ],
    )
    def k(h_hbm, r_hbm, t_hbm, head_hbm, rel_hbm, rat_hbm, out_hbm,
          hi, ri, ti, rel_t, rat_t, sum_t, rows, ob, sem0, sem1, sem_o):
        wid = lax.axis_index("s") * NC + lax.axis_index("c")
        base = wid * BW
        pltpu.sync_copy(h_hbm.at[wid], hi)
        pltpu.sync_copy(r_hbm.at[wid], ri)
        pltpu.sync_copy(t_hbm.at[wid], ti)
        pltpu.sync_copy(rel_hbm, rel_t)
        pltpu.sync_copy(rat_hbm, rat_t)

        for rr in range(3):
            for tt in range(5):
                for dd in range(D // L):
                    sl = pl.ds(dd * L, L)
                    sum_t[rr * 5 + tt, sl] = rel_t[rr, sl] + rat_t[tt, sl]

        sems = (sem0, sem1)

        def issue(p, b):
            def chunk(g, carry):
                hvec = hi[pl.ds(p * CW + g * L, L)]
                for il in range(L):
                    pltpu.async_copy(
                        head_hbm.at[hvec[il]],
                        rows.at[b, g * L + il],
                        sems[b],
                    )
                return carry

            lax.fori_loop(0, CW // L, chunk, 0)

        def drain(b):
            for _ in range(CW):
                pltpu.make_async_copy(
                    head_hbm.at[0], rows.at[b, 0], sems[b]
                ).wait()

        def compute(p, b):
            def chunk(g, carry):
                off = p * CW + g * L
                tvec = ti[pl.ds(off, L)]
                cvec = ri[pl.ds(off, L)] * 5 + tvec
                for il in range(L):
                    i = g * L + il
                    t_s = tvec[il]
                    c_s = cvec[il]
                    for dd in range(D // L):
                        sl = pl.ds(dd * L, L)
                        ob[b, i, 0, sl] = rows[b, i, sl] + rat_t[t_s, sl]
                        ob[b, i, 1, sl] = sum_t[c_s, sl]
                return carry

            lax.fori_loop(0, CW // L, chunk, 0)

        stores = [None, None]
        issue(0, 0)
        for p in range(P):
            b = p % 2
            drain(b)
            if p + 1 < P:
                issue(p + 1, 1 - b)
            if stores[b] is not None:
                stores[b].wait()
            compute(p, b)
            stores[b] = pltpu.async_copy(
                ob.at[b], out_hbm.at[pl.ds(base + p * CW, CW)], sem_o)
        for st in stores:
            if st is not None:
                st.wait()

    return k


def kernel(h, r, t, head_table, relation_table, rating_table):
    B = h.shape[0]
    info = plsc.get_sparse_core_info()
    NW = info.num_cores * info.num_subcores
    h2 = h.reshape(NW, B // NW).astype(jnp.int32)
    r2 = r.reshape(NW, B // NW).astype(jnp.int32)
    t2 = t.reshape(NW, B // NW).astype(jnp.int32)
    k = _build(B)
    return k(h2, r2, t2, head_table, relation_table, rating_table)


# speedup vs baseline: 1.5873x; 1.5873x over previous
import functools

import jax
import jax.numpy as jnp
from jax import lax
from jax.experimental import pallas as pl
from jax.experimental.pallas import tpu as pltpu
from jax.experimental.pallas import tpu_sc as plsc

D = 64
L = 16
BLK = 128
CH = 16
NSLOT = 4


def _build(B, V):
    info = plsc.get_sparse_core_info()
    NC, NS = info.num_cores, info.num_subcores
    NW = NC * NS
    BW = B // NW
    NCH = BW // CH
    TAIL0 = (V // BLK) * BLK
    TAILN = V - TAIL0
    CLAMP = TAIL0 - BLK

    mesh = plsc.VectorSubcoreMesh(core_axis_name="c", subcore_axis_name="s")

    scratch = [
        pltpu.VMEM((BW,), jnp.int32),
        pltpu.VMEM((BW,), jnp.int32),
        pltpu.VMEM((BW,), jnp.int32),
        pltpu.VMEM((3, D), jnp.float32),
        pltpu.VMEM((5, D), jnp.float32),
        pltpu.VMEM((15, D), jnp.float32),
        pltpu.VMEM((NSLOT, 2, D, BLK), jnp.float32),
        pltpu.VMEM((2, CH, 2, D), jnp.float32),
        pltpu.SemaphoreType.DMA,
        pltpu.SemaphoreType.DMA,
        pltpu.SemaphoreType.DMA,
        pltpu.SemaphoreType.DMA,
        pltpu.SemaphoreType.DMA,
    ]
    if TAILN:
        scratch.insert(6, pltpu.VMEM((D, TAILN), jnp.float32))

    @functools.partial(
        pl.kernel,
        mesh=mesh,
        compiler_params=pltpu.CompilerParams(needs_layout_passes=False),
        out_type=jax.ShapeDtypeStruct((B, 2, D), jnp.float32),
        scratch_types=scratch,
    )
    def k(h_hbm, r_hbm, t_hbm, headT_hbm, rel_hbm, rat_hbm, out_hbm,
          hi, ri, ti, rel_t, rat_t, sum_t, *rest):
        if TAILN:
            tail_t, blocks, ob, s0, s1, s2, s3, sem_o = rest
        else:
            blocks, ob, s0, s1, s2, s3, sem_o = rest
            tail_t = None
        sems = (s0, s1, s2, s3)
        wid = lax.axis_index("s") * NC + lax.axis_index("c")
        base = wid * BW
        pltpu.sync_copy(h_hbm.at[wid], hi)
        pltpu.sync_copy(r_hbm.at[wid], ri)
        pltpu.sync_copy(t_hbm.at[wid], ti)
        pltpu.sync_copy(rel_hbm, rel_t)
        pltpu.sync_copy(rat_hbm, rat_t)
        if TAILN:
            pltpu.sync_copy(headT_hbm.at[:, pl.ds(TAIL0, TAILN)], tail_t)

        for rr in range(3):
            for tt in range(5):
                for dd in range(D // L):
                    sl = pl.ds(dd * L, L)
                    sum_t[rr * 5 + tt, sl] = rel_t[rr, sl] + rat_t[tt, sl]

        didx = [lax.iota(jnp.int32, L) + dd * L for dd in range(D // L)]

        def chunk(c, carry):
            off = c * CH
            band = lax.bitwise_and(c, 1)
            hvec = hi[pl.ds(off, L)]
            tvec = ti[pl.ds(off, L)]
            cvec = ri[pl.ds(off, L)] * 5 + tvec
            cbv = jnp.minimum(
                lax.shift_right_logical(hvec, 7) * BLK,
                jnp.full((L,), CLAMP, jnp.int32))
            colv = jnp.minimum(hvec - cbv, jnp.full((L,), BLK - 1, jnp.int32))

            def issue(gq, slot):
                for j in range(2):
                    il = gq * 2 + j
                    cb = pl.multiple_of(cbv[il], BLK)
                    pltpu.async_copy(
                        headT_hbm.at[:, pl.ds(cb, BLK)],
                        blocks.at[slot, j],
                        sems[slot],
                    )

            def drain(slot):
                for _ in range(2):
                    pltpu.make_async_copy(
                        headT_hbm.at[:, pl.ds(0, BLK)],
                        blocks.at[slot, 0],
                        sems[slot],
                    ).wait()

            def compute(gq, slot):
                for j in range(2):
                    il = gq * 2 + j
                    t_s = tvec[il]
                    c_s = cvec[il]
                    col_b = jnp.broadcast_to(colv[il], (L,))
                    for dd in range(D // L):
                        sl = pl.ds(dd * L, L)
                        hv = plsc.load_gather(
                            blocks.at[slot, j], [didx[dd], col_b])
                        ob[band, il, 0, sl] = hv + rat_t[t_s, sl]
                        ob[band, il, 1, sl] = sum_t[c_s, sl]
                    if TAILN:
                        h_s = hvec[il]

                        @pl.when(h_s >= TAIL0)
                        def _fix():
                            tcol = jnp.broadcast_to(h_s - TAIL0, (L,))
                            for dd in range(D // L):
                                sl = pl.ds(dd * L, L)
                                tv = plsc.load_gather(
                                    tail_t, [didx[dd], tcol])
                                ob[band, il, 0, sl] = tv + rat_t[t_s, sl]

            for gq in range(NSLOT):
                issue(gq, gq)
            for gq in range(CH // 2):
                slot = gq % NSLOT
                drain(slot)
                compute(gq, slot)
                if gq + NSLOT < CH // 2:
                    issue(gq + NSLOT, slot)

            @pl.when(c >= 2)
            def _ostore_drain():
                pltpu.make_async_copy(
                    out_hbm.at[pl.ds(0, CH)], ob.at[band], sem_o
                ).wait()

            pltpu.async_copy(
                ob.at[band], out_hbm.at[pl.ds(base + off, CH)], sem_o)
            return carry

        lax.fori_loop(0, NCH, chunk, 0)
        for _ in range(2):
            pltpu.make_async_copy(
                out_hbm.at[pl.ds(0, CH)], ob.at[0], sem_o
            ).wait()

    return k


def kernel(h, r, t, head_table, relation_table, rating_table):
    B = h.shape[0]
    V = head_table.shape[0]
    info = plsc.get_sparse_core_info()
    NW = info.num_cores * info.num_subcores
    h2 = h.reshape(NW, B // NW).astype(jnp.int32)
    r2 = r.reshape(NW, B // NW).astype(jnp.int32)
    t2 = t.reshape(NW, B // NW).astype(jnp.int32)
    k = _build(B, V)
    return k(h2, r2, t2, head_table.T, relation_table, rating_table)


# speedup vs baseline: 1.6975x; 1.0694x over previous
import functools

import jax
import jax.numpy as jnp
from jax import lax
from jax.experimental import pallas as pl
from jax.experimental.pallas import tpu as pltpu
from jax.experimental.pallas import tpu_sc as plsc

D = 64
L = 16
BLK = 128
CH = 32
NSLOT = 6


def _build(B, V):
    info = plsc.get_sparse_core_info()
    NC, NS = info.num_cores, info.num_subcores
    NW = NC * NS
    BW = B // NW
    NCH = BW // CH
    TAIL0 = (V // BLK) * BLK
    TAILN = V - TAIL0
    CLAMP = TAIL0 - BLK

    mesh = plsc.VectorSubcoreMesh(core_axis_name="c", subcore_axis_name="s")

    scratch = [
        pltpu.VMEM((BW,), jnp.int32),
        pltpu.VMEM((BW,), jnp.int32),
        pltpu.VMEM((BW,), jnp.int32),
        pltpu.VMEM((3, D), jnp.float32),
        pltpu.VMEM((5, D), jnp.float32),
        pltpu.VMEM((15, D), jnp.float32),
        pltpu.VMEM((NSLOT, 2, D, BLK), jnp.float32),
        pltpu.VMEM((2, CH, 2, D), jnp.float32),
    ] + [pltpu.SemaphoreType.DMA] * (NSLOT + 1)
    if TAILN:
        scratch.insert(6, pltpu.VMEM((D, TAILN), jnp.float32))

    @functools.partial(
        pl.kernel,
        mesh=mesh,
        compiler_params=pltpu.CompilerParams(needs_layout_passes=False),
        out_type=jax.ShapeDtypeStruct((B, 2, D), jnp.float32),
        scratch_types=scratch,
    )
    def k(h_hbm, r_hbm, t_hbm, headT_hbm, rel_hbm, rat_hbm, out_hbm,
          hi, ri, ti, rel_t, rat_t, sum_t, *rest):
        if TAILN:
            tail_t = rest[0]
            rest = rest[1:]
        else:
            tail_t = None
        blocks, ob = rest[0], rest[1]
        sems = rest[2:2 + NSLOT]
        sem_o = rest[2 + NSLOT]
        wid = lax.axis_index("s") * NC + lax.axis_index("c")
        base = wid * BW
        pltpu.sync_copy(h_hbm.at[wid], hi)
        pltpu.sync_copy(r_hbm.at[wid], ri)
        pltpu.sync_copy(t_hbm.at[wid], ti)
        pltpu.sync_copy(rel_hbm, rel_t)
        pltpu.sync_copy(rat_hbm, rat_t)
        if TAILN:
            pltpu.sync_copy(headT_hbm.at[:, pl.ds(TAIL0, TAILN)], tail_t)

        for rr in range(3):
            for tt in range(5):
                for dd in range(D // L):
                    sl = pl.ds(dd * L, L)
                    sum_t[rr * 5 + tt, sl] = rel_t[rr, sl] + rat_t[tt, sl]

        didx = [lax.iota(jnp.int32, L) + dd * L for dd in range(D // L)]

        def chunk(c, carry):
            off = c * CH
            band = lax.bitwise_and(c, 1)
            hv2, tv2, cv2, cb2, co2 = [], [], [], [], []
            for half in range(CH // L):
                hvec = hi[pl.ds(off + half * L, L)]
                tvec = ti[pl.ds(off + half * L, L)]
                cvec = ri[pl.ds(off + half * L, L)] * 5 + tvec
                cbv = jnp.minimum(
                    lax.shift_right_logical(hvec, 7) * BLK,
                    jnp.full((L,), CLAMP, jnp.int32))
                colv = jnp.minimum(
                    hvec - cbv, jnp.full((L,), BLK - 1, jnp.int32))
                hv2.append(hvec)
                tv2.append(tvec)
                cv2.append(cvec)
                cb2.append(cbv)
                co2.append(colv)

            def issue(gq, slot):
                for j in range(2):
                    il = gq * 2 + j
                    cb = pl.multiple_of(cb2[il // L][il % L], BLK)
                    pltpu.async_copy(
                        headT_hbm.at[:, pl.ds(cb, BLK)],
                        blocks.at[slot, j],
                        sems[slot],
                    )

            def drain(slot):
                for _ in range(2):
                    pltpu.make_async_copy(
                        headT_hbm.at[:, pl.ds(0, BLK)],
                        blocks.at[slot, 0],
                        sems[slot],
                    ).wait()

            def compute(gq, slot):
                for j in range(2):
                    il = gq * 2 + j
                    t_s = tv2[il // L][il % L]
                    c_s = cv2[il // L][il % L]
                    col_b = jnp.broadcast_to(co2[il // L][il % L], (L,))
                    for dd in range(D // L):
                        sl = pl.ds(dd * L, L)
                        hv = plsc.load_gather(
                            blocks.at[slot, j], [didx[dd], col_b])
                        ob[band, il, 0, sl] = hv + rat_t[t_s, sl]
                        ob[band, il, 1, sl] = sum_t[c_s, sl]
                    if TAILN:
                        h_s = hv2[il // L][il % L]

                        @pl.when(h_s >= TAIL0)
                        def _fix():
                            tcol = jnp.broadcast_to(h_s - TAIL0, (L,))
                            for dd in range(D // L):
                                sl = pl.ds(dd * L, L)
                                tv = plsc.load_gather(
                                    tail_t, [didx[dd], tcol])
                                ob[band, il, 0, sl] = tv + rat_t[t_s, sl]

            ngroups = CH // 2
            for gq in range(NSLOT):
                issue(gq, gq)
            for gq in range(ngroups):
                slot = gq % NSLOT
                drain(slot)
                compute(gq, slot)
                if gq + NSLOT < ngroups:
                    issue(gq + NSLOT, slot)

            @pl.when(c >= 2)
            def _ostore_drain():
                pltpu.make_async_copy(
                    out_hbm.at[pl.ds(0, CH)], ob.at[band], sem_o
                ).wait()

            pltpu.async_copy(
                ob.at[band], out_hbm.at[pl.ds(base + off, CH)], sem_o)
            return carry

        lax.fori_loop(0, NCH, chunk, 0)
        for _ in range(2):
            pltpu.make_async_copy(
                out_hbm.at[pl.ds(0, CH)], ob.at[0], sem_o
            ).wait()

    return k


def kernel(h, r, t, head_table, relation_table, rating_table):
    B = h.shape[0]
    V = head_table.shape[0]
    info = plsc.get_sparse_core_info()
    NW = info.num_cores * info.num_subcores
    h2 = h.reshape(NW, B // NW).astype(jnp.int32)
    r2 = r.reshape(NW, B // NW).astype(jnp.int32)
    t2 = t.reshape(NW, B // NW).astype(jnp.int32)
    k = _build(B, V)
    return k(h2, r2, t2, head_table.T, relation_table, rating_table)
